# Initial kernel scaffold; baseline (speedup 1.0000x reference)
import jax
import jax.numpy as jnp
from jax.experimental import pallas as pl


def kernel(x, y):
    raise NotImplementedError("write your pallas kernel here")


# speedup vs baseline: 3.9918x; 3.9918x over previous
import jax
import jax.numpy as jnp
from jax.experimental import pallas as pl
from jax.experimental.pallas import tpu as pltpu

_NPROJ = 32
_R = 1024
_C = 128
_N = _R * _C


def _cmp_exchange(v, j, k, rows, lanes):
    if j >= _R:
        dc = j // _R
        upper = (lanes & dc) != 0
        w = jnp.where(upper, jnp.roll(v, dc, axis=1), jnp.roll(v, -dc, axis=1))
    else:
        upper = (rows & j) != 0
        w = jnp.where(upper, jnp.roll(v, j, axis=0), jnp.roll(v, -j, axis=0))
    if k >= _R:
        asc = (lanes & (k // _R)) == 0
    else:
        asc = (rows & k) == 0
    keep_min = jnp.logical_xor(upper, asc)
    return jnp.where(keep_min, jnp.minimum(v, w), jnp.maximum(v, w))


def _bitonic_sort(v):
    rows = jax.lax.broadcasted_iota(jnp.int32, (_R, _C), 0)
    lanes = jax.lax.broadcasted_iota(jnp.int32, (_R, _C), 1)
    k = 2
    while k <= _N:
        j = k // 2
        while j >= 1:
            v = _cmp_exchange(v, j, k, rows, lanes)
            j //= 2
        k *= 2
    return v


def _proj_kernel(x_ref, y_ref, d_ref, xo_ref, yo_ref):
    d = d_ref[...]
    dn = (((1,), (1,)), ((), ()))
    xo_ref[...] = jax.lax.dot_general(d, x_ref[...], dn,
                                      preferred_element_type=jnp.float32)
    yo_ref[...] = jax.lax.dot_general(d, y_ref[...], dn,
                                      preferred_element_type=jnp.float32)


def _swd_kernel(xp_ref, yp_ref, o_ref):
    xs = _bitonic_sort(xp_ref[0])
    ys = _bitonic_sort(yp_ref[0])
    part = jnp.sum((xs - ys) ** 2)

    @pl.when(pl.program_id(0) == 0)
    def _():
        o_ref[...] = jnp.zeros_like(o_ref)

    o_ref[...] += part.reshape(1, 1)


def kernel(x, y):
    B, N, D = x.shape
    total = B * N
    xf = x.reshape(total, D)
    yf = y.reshape(total, D)

    dkey = jax.random.key(1234)
    dirs = jax.random.normal(dkey, (_NPROJ, D), dtype=x.dtype)
    dirs = dirs / jnp.linalg.norm(dirs, axis=1, keepdims=True)

    chunk = 16384
    nsteps = total // chunk
    xp, yp = pl.pallas_call(
        _proj_kernel,
        grid=(nsteps,),
        in_specs=[
            pl.BlockSpec((chunk, D), lambda i: (i, 0)),
            pl.BlockSpec((chunk, D), lambda i: (i, 0)),
            pl.BlockSpec((_NPROJ, D), lambda i: (0, 0)),
        ],
        out_specs=[
            pl.BlockSpec((_NPROJ, chunk), lambda i: (0, i)),
            pl.BlockSpec((_NPROJ, chunk), lambda i: (0, i)),
        ],
        out_shape=[
            jax.ShapeDtypeStruct((_NPROJ, total), jnp.float32),
            jax.ShapeDtypeStruct((_NPROJ, total), jnp.float32),
        ],
    )(xf, yf, dirs)

    xp3 = xp.reshape(_NPROJ, _R, _C)
    yp3 = yp.reshape(_NPROJ, _R, _C)

    acc = pl.pallas_call(
        _swd_kernel,
        grid=(_NPROJ,),
        in_specs=[
            pl.BlockSpec((1, _R, _C), lambda p: (p, 0, 0)),
            pl.BlockSpec((1, _R, _C), lambda p: (p, 0, 0)),
        ],
        out_specs=pl.BlockSpec((1, 1), lambda p: (0, 0)),
        out_shape=jax.ShapeDtypeStruct((1, 1), jnp.float32),
    )(xp3, yp3)

    return acc[0, 0] / (_NPROJ * _N)


# speedup vs baseline: 4.0440x; 1.0131x over previous
import jax
import jax.numpy as jnp
from jax.experimental import pallas as pl
from jax.experimental.pallas import tpu as pltpu

_NPROJ = 32
_R = 1024
_C = 128
_N = _R * _C


def _cmp_exchange(v, j, k, rows, lanes):
    if 8 <= j < _R:
        m = _R // (2 * j)
        a = v.reshape(m, 2, j, _C)
        lo = a[:, 0]
        hi = a[:, 1]
        mn = jnp.minimum(lo, hi)
        mx = jnp.maximum(lo, hi)
        if k >= _R:
            asc = (jax.lax.broadcasted_iota(jnp.int32, (m, j, _C), 2)
                   & (k // _R)) == 0
        else:
            bi = jax.lax.broadcasted_iota(jnp.int32, (m, j, _C), 0)
            asc = ((bi * 2 * j) & k) == 0
        new_lo = jnp.where(asc, mn, mx)
        new_hi = jnp.where(asc, mx, mn)
        return jnp.concatenate([new_lo[:, None], new_hi[:, None]],
                               axis=1).reshape(_R, _C)
    if j >= _R:
        dc = j // _R
        upper = (lanes & dc) != 0
        w = jnp.where(upper, jnp.roll(v, dc, axis=1), jnp.roll(v, -dc, axis=1))
    else:
        upper = (rows & j) != 0
        w = jnp.where(upper, jnp.roll(v, j, axis=0), jnp.roll(v, -j, axis=0))
    if k >= _R:
        asc = (lanes & (k // _R)) == 0
    else:
        asc = (rows & k) == 0
    keep_min = jnp.logical_xor(upper, asc)
    return jnp.where(keep_min, jnp.minimum(v, w), jnp.maximum(v, w))


def _bitonic_sort(v):
    rows = jax.lax.broadcasted_iota(jnp.int32, (_R, _C), 0)
    lanes = jax.lax.broadcasted_iota(jnp.int32, (_R, _C), 1)
    k = 2
    while k <= _N:
        j = k // 2
        while j >= 1:
            v = _cmp_exchange(v, j, k, rows, lanes)
            j //= 2
        k *= 2
    return v


def _proj_kernel(x_ref, y_ref, d_ref, xo_ref, yo_ref):
    d = d_ref[...]
    dn = (((1,), (1,)), ((), ()))
    xo_ref[...] = jax.lax.dot_general(d, x_ref[...], dn,
                                      preferred_element_type=jnp.float32)
    yo_ref[...] = jax.lax.dot_general(d, y_ref[...], dn,
                                      preferred_element_type=jnp.float32)


def _swd_kernel(xp_ref, yp_ref, o_ref):
    xs = _bitonic_sort(xp_ref[0])
    ys = _bitonic_sort(yp_ref[0])
    part = jnp.sum((xs - ys) ** 2)

    @pl.when(pl.program_id(0) == 0)
    def _():
        o_ref[...] = jnp.zeros_like(o_ref)

    o_ref[...] += part.reshape(1, 1)


def kernel(x, y):
    B, N, D = x.shape
    total = B * N
    xf = x.reshape(total, D)
    yf = y.reshape(total, D)

    dkey = jax.random.key(1234)
    dirs = jax.random.normal(dkey, (_NPROJ, D), dtype=x.dtype)
    dirs = dirs / jnp.linalg.norm(dirs, axis=1, keepdims=True)

    chunk = 16384
    nsteps = total // chunk
    xp, yp = pl.pallas_call(
        _proj_kernel,
        grid=(nsteps,),
        in_specs=[
            pl.BlockSpec((chunk, D), lambda i: (i, 0)),
            pl.BlockSpec((chunk, D), lambda i: (i, 0)),
            pl.BlockSpec((_NPROJ, D), lambda i: (0, 0)),
        ],
        out_specs=[
            pl.BlockSpec((_NPROJ, chunk), lambda i: (0, i)),
            pl.BlockSpec((_NPROJ, chunk), lambda i: (0, i)),
        ],
        out_shape=[
            jax.ShapeDtypeStruct((_NPROJ, total), jnp.float32),
            jax.ShapeDtypeStruct((_NPROJ, total), jnp.float32),
        ],
    )(xf, yf, dirs)

    xp3 = xp.reshape(_NPROJ, _R, _C)
    yp3 = yp.reshape(_NPROJ, _R, _C)

    acc = pl.pallas_call(
        _swd_kernel,
        grid=(_NPROJ,),
        in_specs=[
            pl.BlockSpec((1, _R, _C), lambda p: (p, 0, 0)),
            pl.BlockSpec((1, _R, _C), lambda p: (p, 0, 0)),
        ],
        out_specs=pl.BlockSpec((1, 1), lambda p: (0, 0)),
        out_shape=jax.ShapeDtypeStruct((1, 1), jnp.float32),
    )(xp3, yp3)

    return acc[0, 0] / (_NPROJ * _N)


# speedup vs baseline: 4.9481x; 1.2236x over previous
import jax
import jax.numpy as jnp
from jax.experimental import pallas as pl
from jax.experimental.pallas import tpu as pltpu

_NPROJ = 32
_R = 1024
_C = 128
_N = _R * _C


def _asc_mask(k, rows, lanes):
    if k >= _R:
        return (lanes & (k // _R)) == 0
    return (rows & k) == 0


def _lane_pass(w, j, k, lanes):
    dc = j // _R
    upper = (lanes & dc) != 0
    partner = jnp.where(upper, jnp.roll(w, dc, axis=2),
                        jnp.roll(w, -dc, axis=2))
    keep_min = jnp.logical_xor(upper, _asc_mask(k, None, lanes))
    return jnp.where(keep_min, jnp.minimum(w, partner),
                     jnp.maximum(w, partner))


def _axis0_pass(w, d, k, rows, lanes):
    g = w.shape[0] // (2 * d)
    tail = w.shape[1:]
    a = w.reshape((g, 2, d) + tail)
    lo, hi = a[:, 0], a[:, 1]
    mn = jnp.minimum(lo, hi)
    mx = jnp.maximum(lo, hi)
    asc = _asc_mask(k, rows.reshape((g, 2, d) + tail)[:, 0],
                    lanes.reshape((g, 2, d) + tail)[:, 0])
    new_lo = jnp.where(asc, mn, mx)
    new_hi = jnp.where(asc, mx, mn)
    return jnp.concatenate([new_lo[:, None], new_hi[:, None]],
                           axis=1).reshape(w.shape)


def _sublane_pass(w, j, k, rows, lanes):
    s = jax.lax.broadcasted_iota(jnp.int32, w.shape, 1)
    upper = (s & j) != 0
    partner = jnp.where(upper, jnp.roll(w, j, axis=1),
                        jnp.roll(w, -j, axis=1))
    keep_min = jnp.logical_xor(upper, _asc_mask(k, rows, lanes))
    return jnp.where(keep_min, jnp.minimum(w, partner),
                     jnp.maximum(w, partner))


def _sort_ref(ref):
    sh_a = (16, 8, _C)
    rows_a_base = (64 * jax.lax.broadcasted_iota(jnp.int32, sh_a, 0)
                   + jax.lax.broadcasted_iota(jnp.int32, sh_a, 1))
    lanes_a = jax.lax.broadcasted_iota(jnp.int32, sh_a, 2)
    sh_b = (8, 8, _C)
    rows_b_base = (8 * jax.lax.broadcasted_iota(jnp.int32, sh_b, 0)
                   + jax.lax.broadcasted_iota(jnp.int32, sh_b, 1))
    lanes_b = jax.lax.broadcasted_iota(jnp.int32, sh_b, 2)

    for ke in range(1, 18):
        k = 1 << ke
        if k >= 128:
            for i in range(8):
                w = ref[:, i]
                rows = rows_a_base + 8 * i
                j = k // 2
                while j >= _R:
                    w = _lane_pass(w, j, k, lanes_a)
                    j //= 2
                for j in (512, 256, 128, 64):
                    if j < k:
                        w = _axis0_pass(w, j // 64, k, rows, lanes_a)
                ref[:, i] = w
        for m in range(16):
            w = ref[m]
            rows = rows_b_base + 64 * m
            for j in (32, 16, 8):
                if j < k:
                    w = _axis0_pass(w, j // 8, k, rows, lanes_b)
            for j in (4, 2, 1):
                if j < k:
                    w = _sublane_pass(w, j, k, rows, lanes_b)
            ref[m] = w


def _proj_kernel(x_ref, y_ref, d_ref, xo_ref, yo_ref):
    d = d_ref[...]
    dn = (((1,), (1,)), ((), ()))
    xo_ref[...] = jax.lax.dot_general(d, x_ref[...], dn,
                                      preferred_element_type=jnp.float32)
    yo_ref[...] = jax.lax.dot_general(d, y_ref[...], dn,
                                      preferred_element_type=jnp.float32)


def _swd_kernel(xp_ref, yp_ref, o_ref, xs_ref, ys_ref):
    xs_ref[...] = xp_ref[0]
    ys_ref[...] = yp_ref[0]
    _sort_ref(xs_ref)
    _sort_ref(ys_ref)
    part = jnp.sum((xs_ref[...] - ys_ref[...]) ** 2)

    @pl.when(pl.program_id(0) == 0)
    def _():
        o_ref[...] = jnp.zeros_like(o_ref)

    o_ref[...] += part.reshape(1, 1)


def kernel(x, y):
    B, N, D = x.shape
    total = B * N
    xf = x.reshape(total, D)
    yf = y.reshape(total, D)

    dkey = jax.random.key(1234)
    dirs = jax.random.normal(dkey, (_NPROJ, D), dtype=x.dtype)
    dirs = dirs / jnp.linalg.norm(dirs, axis=1, keepdims=True)

    chunk = 16384
    nsteps = total // chunk
    xp, yp = pl.pallas_call(
        _proj_kernel,
        grid=(nsteps,),
        in_specs=[
            pl.BlockSpec((chunk, D), lambda i: (i, 0)),
            pl.BlockSpec((chunk, D), lambda i: (i, 0)),
            pl.BlockSpec((_NPROJ, D), lambda i: (0, 0)),
        ],
        out_specs=[
            pl.BlockSpec((_NPROJ, chunk), lambda i: (0, i)),
            pl.BlockSpec((_NPROJ, chunk), lambda i: (0, i)),
        ],
        out_shape=[
            jax.ShapeDtypeStruct((_NPROJ, total), jnp.float32),
            jax.ShapeDtypeStruct((_NPROJ, total), jnp.float32),
        ],
    )(xf, yf, dirs)

    xp5 = xp.reshape(_NPROJ, 16, 8, 8, _C)
    yp5 = yp.reshape(_NPROJ, 16, 8, 8, _C)

    acc = pl.pallas_call(
        _swd_kernel,
        grid=(_NPROJ,),
        in_specs=[
            pl.BlockSpec((1, 16, 8, 8, _C), lambda p: (p, 0, 0, 0, 0)),
            pl.BlockSpec((1, 16, 8, 8, _C), lambda p: (p, 0, 0, 0, 0)),
        ],
        out_specs=pl.BlockSpec((1, 1), lambda p: (0, 0)),
        out_shape=jax.ShapeDtypeStruct((1, 1), jnp.float32),
        scratch_shapes=[
            pltpu.VMEM((16, 8, 8, _C), jnp.float32),
            pltpu.VMEM((16, 8, 8, _C), jnp.float32),
        ],
    )(xp5, yp5)

    return acc[0, 0] / (_NPROJ * _N)


# speedup vs baseline: 5.1477x; 1.0403x over previous
import jax
import jax.numpy as jnp
from jax.experimental import pallas as pl
from jax.experimental.pallas import tpu as pltpu

_NPROJ = 32
_R = 1024
_C = 128
_N = _R * _C


def _asc_mask(k, rows, lanes):
    if k >= _R:
        return (lanes & (k // _R)) == 0
    return (rows & k) == 0


def _lane_pass(w, j, k, lanes):
    dc = j // _R
    upper = (lanes & dc) != 0
    partner = jnp.where(upper, pltpu.roll(w, dc, 2),
                        pltpu.roll(w, w.shape[2] - dc, 2))
    keep_min = jnp.logical_xor(upper, _asc_mask(k, None, lanes))
    return jnp.where(keep_min, jnp.minimum(w, partner),
                     jnp.maximum(w, partner))


def _axis0_pass(w, d, k, rows, lanes):
    g = w.shape[0] // (2 * d)
    tail = w.shape[1:]
    a = w.reshape((g, 2, d) + tail)
    lo, hi = a[:, 0], a[:, 1]
    mn = jnp.minimum(lo, hi)
    mx = jnp.maximum(lo, hi)
    asc = _asc_mask(k, rows.reshape((g, 2, d) + tail)[:, 0],
                    lanes.reshape((g, 2, d) + tail)[:, 0])
    new_lo = jnp.where(asc, mn, mx)
    new_hi = jnp.where(asc, mx, mn)
    return jnp.concatenate([new_lo[:, None], new_hi[:, None]],
                           axis=1).reshape(w.shape)


def _sublane_pass(w, j, k, rows, lanes):
    s = jax.lax.broadcasted_iota(jnp.int32, w.shape, 1)
    upper = (s & j) != 0
    partner = jnp.where(upper, pltpu.roll(w, j, 1),
                        pltpu.roll(w, w.shape[1] - j, 1))
    keep_min = jnp.logical_xor(upper, _asc_mask(k, rows, lanes))
    return jnp.where(keep_min, jnp.minimum(w, partner),
                     jnp.maximum(w, partner))


def _sort_refs(chains):
    sh_a = (16, 8, _C)
    rows_a_base = (64 * jax.lax.broadcasted_iota(jnp.int32, sh_a, 0)
                   + jax.lax.broadcasted_iota(jnp.int32, sh_a, 1))
    lanes_a = jax.lax.broadcasted_iota(jnp.int32, sh_a, 2)
    sh_b = (8, 8, _C)
    rows_b_base = (8 * jax.lax.broadcasted_iota(jnp.int32, sh_b, 0)
                   + jax.lax.broadcasted_iota(jnp.int32, sh_b, 1))
    lanes_b = jax.lax.broadcasted_iota(jnp.int32, sh_b, 2)

    cur = [c[0] for c in chains]
    nxt = [c[1] for c in chains]

    def flip(ci):
        a, b = chains[ci][1], chains[ci][2]
        cur[ci] = nxt[ci]
        nxt[ci] = b if nxt[ci] is a else a

    for ke in range(1, 18):
        k = 1 << ke
        if k >= 128:
            for i in range(8):
                rows = rows_a_base + 8 * i
                for ci in range(len(chains)):
                    w = cur[ci][:, i]
                    j = k // 2
                    while j >= _R:
                        w = _lane_pass(w, j, k, lanes_a)
                        j //= 2
                    for j in (512, 256, 128, 64):
                        if j < k:
                            w = _axis0_pass(w, j // 64, k, rows, lanes_a)
                    nxt[ci][:, i] = w
            for ci in range(len(chains)):
                flip(ci)
        for m in range(16):
            rows = rows_b_base + 64 * m
            for ci in range(len(chains)):
                w = cur[ci][m]
                for j in (32, 16, 8):
                    if j < k:
                        w = _axis0_pass(w, j // 8, k, rows, lanes_b)
                for j in (4, 2, 1):
                    if j < k:
                        w = _sublane_pass(w, j, k, rows, lanes_b)
                nxt[ci][m] = w
        for ci in range(len(chains)):
            flip(ci)
    return cur


def _proj_kernel(x_ref, y_ref, d_ref, xo_ref, yo_ref):
    d = d_ref[...]
    dn = (((1,), (1,)), ((), ()))
    xp = jax.lax.dot_general(d, x_ref[...], dn,
                             preferred_element_type=jnp.float32)
    yp = jax.lax.dot_general(d, y_ref[...], dn,
                             preferred_element_type=jnp.float32)
    xo_ref[...] = xp.reshape(xo_ref.shape)
    yo_ref[...] = yp.reshape(yo_ref.shape)


def _swd_kernel(xp_ref, yp_ref, o_ref, xa_ref, xb_ref, ya_ref, yb_ref):
    xs, ys = _sort_refs([[xp_ref.at[0], xa_ref, xb_ref],
                         [yp_ref.at[0], ya_ref, yb_ref]])
    part = jnp.sum((xs[...] - ys[...]) ** 2)
    o_ref[...] = part.reshape(1, 1, 1)


def kernel(x, y):
    B, N, D = x.shape
    total = B * N
    xf = x.reshape(total, D)
    yf = y.reshape(total, D)

    dkey = jax.random.key(1234)
    dirs = jax.random.normal(dkey, (_NPROJ, D), dtype=x.dtype)
    dirs = dirs / jnp.linalg.norm(dirs, axis=1, keepdims=True)

    chunk = 16384
    nsteps = total // chunk
    xp, yp = pl.pallas_call(
        _proj_kernel,
        grid=(nsteps,),
        in_specs=[
            pl.BlockSpec((chunk, D), lambda i: (i, 0)),
            pl.BlockSpec((chunk, D), lambda i: (i, 0)),
            pl.BlockSpec((_NPROJ, D), lambda i: (0, 0)),
        ],
        out_specs=[
            pl.BlockSpec((_NPROJ, 2, 8, 8, _C), lambda i: (0, i, 0, 0, 0)),
            pl.BlockSpec((_NPROJ, 2, 8, 8, _C), lambda i: (0, i, 0, 0, 0)),
        ],
        out_shape=[
            jax.ShapeDtypeStruct((_NPROJ, 16, 8, 8, _C), jnp.float32),
            jax.ShapeDtypeStruct((_NPROJ, 16, 8, 8, _C), jnp.float32),
        ],
    )(xf, yf, dirs)
    xp5, yp5 = xp, yp

    acc = pl.pallas_call(
        _swd_kernel,
        grid=(_NPROJ,),
        in_specs=[
            pl.BlockSpec((1, 16, 8, 8, _C), lambda p: (p, 0, 0, 0, 0)),
            pl.BlockSpec((1, 16, 8, 8, _C), lambda p: (p, 0, 0, 0, 0)),
        ],
        out_specs=pl.BlockSpec((1, 1, 1), lambda p: (p, 0, 0)),
        out_shape=jax.ShapeDtypeStruct((_NPROJ, 1, 1), jnp.float32),
        scratch_shapes=[
            pltpu.VMEM((16, 8, 8, _C), jnp.float32),
            pltpu.VMEM((16, 8, 8, _C), jnp.float32),
            pltpu.VMEM((16, 8, 8, _C), jnp.float32),
            pltpu.VMEM((16, 8, 8, _C), jnp.float32),
        ],
        compiler_params=pltpu.CompilerParams(
            dimension_semantics=("parallel",)),
    )(xp5, yp5)

    return jnp.sum(acc) / (_NPROJ * _N)
